# baseline (speedup 1.0000x reference)
import functools

import jax
import jax.numpy as jnp
from jax import lax
from jax.experimental import pallas as pl
from jax.experimental.pallas import tpu as pltpu
from jax.experimental.pallas import tpu_sc as plsc

F32 = jnp.float32
I32 = jnp.int32

N = 100000
G = 128
H = 64
NPAD = 100352
BLK = 2048
NB = NPAD // BLK
RPT = NPAD // 16
EPAD = 1605632
CHUNK = 512
EPC = EPAD // 16
EPW = EPAD // 32

_MESH = dict(core_axis_name="c", subcore_axis_name="s")
_SC_PARAMS = pltpu.CompilerParams(use_tc_tiling_on_sc=False)


def _zero_vmem(ref, nrows):
    z = jnp.zeros((16,), F32)

    def body(i, _):
        ref[i, :] = z
        return 0

    lax.fori_loop(0, nrows, body, 0)


def _zero_vmem_1d(ref, n16):
    z = jnp.zeros((16,), F32)

    def body(i, _):
        ref[pl.ds(i * 16, 16)] = z
        return 0

    lax.fori_loop(0, n16, body, 0)


def _fill_ones_1d(ref, n16):
    o = jnp.ones((16,), F32)

    def body(i, _):
        ref[pl.ds(i * 16, 16)] = o
        return 0

    lax.fori_loop(0, n16, body, 0)


def _deg_body(dst1, out, onesv, idxv, zv, shared, sem):
    core = lax.axis_index("c")
    sub = lax.axis_index("s")
    wid = core * 16 + sub
    row0 = sub * RPT

    _fill_ones_1d(onesv, CHUNK // 16)
    _zero_vmem_1d(zv, RPT // 16)
    pltpu.sync_copy(zv, shared.at[pl.ds(row0, RPT)])
    plsc.subcore_barrier()

    def chunk(j, _):
        e0 = wid * EPW + j * CHUNK
        pltpu.sync_copy(dst1.at[pl.ds(e0, CHUNK)], idxv)
        pltpu.async_copy(onesv, shared.at[idxv], sem, add=True).wait()
        return 0

    lax.fori_loop(0, EPW // CHUNK, chunk, 0)
    plsc.subcore_barrier()
    pltpu.async_copy(shared.at[pl.ds(row0, RPT)],
                     out.at[core, pl.ds(row0, RPT)], sem).wait()


@functools.partial(
    pl.kernel,
    out_type=jax.ShapeDtypeStruct((2, NPAD), F32),
    mesh=plsc.VectorSubcoreMesh(**_MESH),
    compiler_params=_SC_PARAMS,
    scratch_types=[
        pltpu.VMEM((CHUNK,), F32),
        pltpu.VMEM((CHUNK,), I32),
        pltpu.VMEM((RPT,), F32),
        pltpu.VMEM_SHARED((NPAD,), F32),
        pltpu.SemaphoreType.DMA,
    ],
)
def _k_deg(dst1, out, onesv, idxv, zv, shared, sem):
    _deg_body(dst1, out, onesv, idxv, zv, shared, sem)


def _rows_body(with_ex, with_denom, rounds, ytab, src1, dst1, ex1, outs,
               srcva, dstva, rowsva, exva, srcvb, dstvb, rowsvb, exvb, zv,
               shared, gsem, ssem, zv2=None, sharedd=None):
    if with_denom:
        out, dpart = outs
    else:
        out = outs
    core = lax.axis_index("c")
    sub = lax.axis_index("s")
    row0 = sub * RPT

    _zero_vmem(zv, RPT // 32)

    def _load_adj(e0, srcv, dstv, off):
        pltpu.sync_copy(src1.at[pl.ds(e0, CHUNK)], srcv)
        pltpu.sync_copy(dst1.at[pl.ds(e0, CHUNK)], dstv)

        def adj(i, _):
            sl = pl.ds(i * 16, 16)
            srcv[sl] = srcv[sl] + off
            return 0

        lax.fori_loop(0, CHUNK // 16, adj, 0)

    def _mul_ex(e0, rowsv, exv):
        pltpu.sync_copy(ex1.at[pl.ds(e0, CHUNK)], exv)

        def mul(i, _):
            e16 = exv[pl.ds(i * 16, 16)]
            base = i * 16
            for t in range(16):
                rowsv[base + t, :] = rowsv[base + t, :] * e16[t]
            return 0

        lax.fori_loop(0, CHUNK // 16, mul, 0)

    for rnd in range(rounds):
        col = core + 2 * rnd
        off = col * NPAD
        first = rnd == 0
        zd = [
            pltpu.async_copy(
                zv, shared.at[pl.ds(row0 + t * (RPT // 32), RPT // 32)],
                gsem)
            for t in range(32)
        ]
        if with_denom and first:
            _zero_vmem_1d(zv2, RPT // 8 // 16)
            zd += [
                pltpu.async_copy(
                    zv2, sharedd.at[pl.ds(row0 + t * (RPT // 8), RPT // 8)],
                    gsem)
                for t in range(8)
            ]
        for d in zd:
            d.wait()
        plsc.subcore_barrier()

        def pair(j, _):
            e0 = sub * EPC + j * (2 * CHUNK)
            e1 = e0 + CHUNK
            _load_adj(e0, srcva, dstva, off)
            ga = pltpu.async_copy(ytab.at[srcva], rowsva, gsem)
            _load_adj(e1, srcvb, dstvb, off)
            ga.wait()
            if with_ex:
                _mul_ex(e0, rowsva, exva)
            sa = pltpu.async_copy(rowsva, shared.at[dstva], ssem, add=True)
            da = (pltpu.async_copy(exva, sharedd.at[dstva], ssem, add=True)
                  if with_denom and first else None)
            gb = pltpu.async_copy(ytab.at[srcvb], rowsvb, gsem)
            gb.wait()
            if with_ex:
                _mul_ex(e1, rowsvb, exvb)
            sb = pltpu.async_copy(rowsvb, shared.at[dstvb], ssem, add=True)
            db = (pltpu.async_copy(exvb, sharedd.at[dstvb], ssem, add=True)
                  if with_denom and first else None)
            for d in (sa, da, sb, db):
                if d is not None:
                    d.wait()
            return 0

        lax.fori_loop(0, EPC // (2 * CHUNK), pair, 0)
        plsc.subcore_barrier()
        dr = [pltpu.async_copy(shared.at[pl.ds(row0, RPT)],
                               out.at[pl.ds(off + row0, RPT)], gsem)]
        if with_denom and first:
            dr.append(pltpu.async_copy(
                sharedd.at[pl.ds(row0, RPT)],
                dpart.at[core, pl.ds(row0, RPT)], gsem))
        for d in dr:
            d.wait()
        plsc.subcore_barrier()


def _make_rows(with_ex, rounds, with_denom=False):
    ncols = 2 * rounds
    scratch = [
        pltpu.VMEM((CHUNK,), I32),
        pltpu.VMEM((CHUNK,), I32),
        pltpu.VMEM((CHUNK, 16), F32),
        pltpu.VMEM((CHUNK,), F32),
        pltpu.VMEM((CHUNK,), I32),
        pltpu.VMEM((CHUNK,), I32),
        pltpu.VMEM((CHUNK, 16), F32),
        pltpu.VMEM((CHUNK,), F32),
        pltpu.VMEM((RPT // 32, 16), F32),
        pltpu.VMEM_SHARED((NPAD, 16), F32),
        pltpu.SemaphoreType.DMA,
        pltpu.SemaphoreType.DMA,
    ]
    if with_denom:
        scratch = scratch + [
            pltpu.VMEM((RPT // 8,), F32),
            pltpu.VMEM_SHARED((NPAD,), F32),
        ]

        @functools.partial(
            pl.kernel,
            out_type=(jax.ShapeDtypeStruct((ncols * NPAD, 16), F32),
                      jax.ShapeDtypeStruct((2, NPAD), F32)),
            mesh=plsc.VectorSubcoreMesh(**_MESH),
            compiler_params=_SC_PARAMS,
            scratch_types=scratch,
        )
        def k(ytab, src1, dst1, ex1, out, dpart, *s):
            _rows_body(True, True, rounds, ytab, src1, dst1, ex1,
                       (out, dpart), *s)
    elif with_ex:
        @functools.partial(
            pl.kernel,
            out_type=jax.ShapeDtypeStruct((ncols * NPAD, 16), F32),
            mesh=plsc.VectorSubcoreMesh(**_MESH),
            compiler_params=_SC_PARAMS,
            scratch_types=scratch,
        )
        def k(ytab, src1, dst1, ex1, out, *s):
            _rows_body(True, False, rounds, ytab, src1, dst1, ex1, out, *s)
    else:
        @functools.partial(
            pl.kernel,
            out_type=jax.ShapeDtypeStruct((ncols * NPAD, 16), F32),
            mesh=plsc.VectorSubcoreMesh(**_MESH),
            compiler_params=_SC_PARAMS,
            scratch_types=scratch,
        )
        def k(ytab, src1, dst1, out, *s):
            _rows_body(False, False, rounds, ytab, src1, dst1, None, out, *s)
    return k


_k_rows_coef = _make_rows(True, 2)
_k_rows_ex = _make_rows(True, 2, with_denom=True)


def _score_body(als, ald, cs, src1, dst1, s1, srcv, dstv, asv, adv, csv, exv,
                gsem):
    core = lax.axis_index("c")
    sub = lax.axis_index("s")
    wid = core * 16 + sub

    def chunk(j, _):
        e0 = wid * EPW + j * CHUNK
        pltpu.sync_copy(src1.at[pl.ds(e0, CHUNK)], srcv)
        pltpu.sync_copy(dst1.at[pl.ds(e0, CHUNK)], dstv)
        gd = [
            pltpu.async_copy(als.at[srcv], asv, gsem),
            pltpu.async_copy(ald.at[dstv], adv, gsem),
            pltpu.async_copy(cs.at[dstv], csv, gsem),
        ]
        for d in gd:
            d.wait()

        def comp(i, _):
            sl = pl.ds(i * 16, 16)
            s = asv[sl] + adv[sl]
            e = jnp.maximum(s, 0.0) + 0.2 * jnp.minimum(s, 0.0)
            exv[sl] = e - csv[sl]
            return 0

        lax.fori_loop(0, CHUNK // 16, comp, 0)
        pltpu.sync_copy(exv, s1.at[pl.ds(e0, CHUNK)])
        return 0

    lax.fori_loop(0, EPW // CHUNK, chunk, 0)


@functools.partial(
    pl.kernel,
    out_type=jax.ShapeDtypeStruct((EPAD,), F32),
    mesh=plsc.VectorSubcoreMesh(**_MESH),
    compiler_params=_SC_PARAMS,
    scratch_types=[
        pltpu.VMEM((CHUNK,), I32),
        pltpu.VMEM((CHUNK,), I32),
        pltpu.VMEM((CHUNK,), F32),
        pltpu.VMEM((CHUNK,), F32),
        pltpu.VMEM((CHUNK,), F32),
        pltpu.VMEM((CHUNK,), F32),
        pltpu.SemaphoreType.DMA,
    ],
)
def _k_score(als, ald, cs, src1, dst1, s1, *s):
    _score_body(als, ald, cs, src1, dst1, s1, *s)


def _norm_body(dinv, src1, dst1, norm1, srcv, dstv, dsv, ddv, nv, gsem):
    core = lax.axis_index("c")
    sub = lax.axis_index("s")
    wid = core * 16 + sub

    def chunk(j, _):
        e0 = wid * EPW + j * CHUNK
        pltpu.sync_copy(src1.at[pl.ds(e0, CHUNK)], srcv)
        pltpu.sync_copy(dst1.at[pl.ds(e0, CHUNK)], dstv)
        gd = [
            pltpu.async_copy(dinv.at[srcv], dsv, gsem),
            pltpu.async_copy(dinv.at[dstv], ddv, gsem),
        ]
        for d in gd:
            d.wait()

        def comp(i, _):
            sl = pl.ds(i * 16, 16)
            nv[sl] = dsv[sl] * ddv[sl]
            return 0

        lax.fori_loop(0, CHUNK // 16, comp, 0)
        pltpu.sync_copy(nv, norm1.at[pl.ds(e0, CHUNK)])
        return 0

    lax.fori_loop(0, EPW // CHUNK, chunk, 0)


@functools.partial(
    pl.kernel,
    out_type=jax.ShapeDtypeStruct((EPAD,), F32),
    mesh=plsc.VectorSubcoreMesh(**_MESH),
    compiler_params=_SC_PARAMS,
    scratch_types=[
        pltpu.VMEM((CHUNK,), I32),
        pltpu.VMEM((CHUNK,), I32),
        pltpu.VMEM((CHUNK,), F32),
        pltpu.VMEM((CHUNK,), F32),
        pltpu.VMEM((CHUNK,), F32),
        pltpu.SemaphoreType.DMA,
    ],
)
def _k_norm(dinv, src1, dst1, norm1, *s):
    _norm_body(dinv, src1, dst1, norm1, *s)


_EXR = EPAD // 128
_EXB = _EXR // NB


def _exp_body(s_ref, e_ref):
    e_ref[...] = jnp.exp(s_ref[...])


_k_exp = pl.pallas_call(
    _exp_body,
    grid=(NB,),
    in_specs=[pl.BlockSpec((_EXB, 128), lambda i: (i, 0))],
    out_specs=pl.BlockSpec((_EXB, 128), lambda i: (i, 0)),
    out_shape=jax.ShapeDtypeStruct((_EXR, 128), F32),
)


def _prep1_body(x_ref, degp_ref, w_ref, ytab_ref, dinv_ref):
    deg = degp_ref[0] + degp_ref[1] + 1.0
    dinv = 1.0 / jnp.sqrt(deg)
    y = jnp.dot(x_ref[...], w_ref[...], preferred_element_type=F32,
                precision=jax.lax.Precision.DEFAULT)
    for c in range(4):
        ytab_ref[c] = y[:, c * 16:(c + 1) * 16]
    dinv_ref[...] = dinv


_k_prep1 = pl.pallas_call(
    _prep1_body,
    grid=(NB,),
    in_specs=[
        pl.BlockSpec((BLK, 32), lambda i: (i, 0)),
        pl.BlockSpec((2, BLK, 1), lambda i: (0, i, 0)),
        pl.BlockSpec((32, 64), lambda i: (0, 0)),
    ],
    out_specs=[
        pl.BlockSpec((4, BLK, 16), lambda i: (0, i, 0)),
        pl.BlockSpec((BLK, 1), lambda i: (i, 0)),
    ],
    out_shape=[
        jax.ShapeDtypeStruct((4, NPAD, 16), F32),
        jax.ShapeDtypeStruct((NPAD, 1), F32),
    ],
)


def _gcn1_body(raw_ref, ytab1_ref, dinv_ref, b1_ref, w2_ref, ytab_ref):
    n2 = dinv_ref[...] * dinv_ref[...]
    hs = []
    for c in range(4):
        pre = raw_ref[c] + n2 * ytab1_ref[c]
        hs.append(jnp.maximum(pre + b1_ref[pl.ds(c * 16, 16)], 0.0))
    h1 = jnp.concatenate(hs, axis=1)
    y = jnp.dot(h1, w2_ref[...], preferred_element_type=F32,
                precision=jax.lax.Precision.DEFAULT)
    for c in range(4):
        ytab_ref[c] = y[:, c * 16:(c + 1) * 16]


_k_gcn1 = pl.pallas_call(
    _gcn1_body,
    grid=(NB,),
    in_specs=[
        pl.BlockSpec((4, BLK, 16), lambda i: (0, i, 0)),
        pl.BlockSpec((4, BLK, 16), lambda i: (0, i, 0)),
        pl.BlockSpec((BLK, 1), lambda i: (i, 0)),
        pl.BlockSpec((64,), lambda i: (0,)),
        pl.BlockSpec((64, 64), lambda i: (0, 0)),
    ],
    out_specs=pl.BlockSpec((4, BLK, 16), lambda i: (0, i, 0)),
    out_shape=jax.ShapeDtypeStruct((4, NPAD, 16), F32),
)


def _gatprep_body(scale_kind, raw_ref, selftab_ref, s_ref, b_ref, w_ref,
                  asrc_ref, adst_ref, xwtab_ref, als_ref, ald_ref, cs_ref):
    acc = jnp.zeros((BLK, 64), F32)
    for c in range(4):
        if scale_kind == "dinv":
            n2 = s_ref[...] * s_ref[...]
            pre = raw_ref[c] + n2 * selftab_ref[c]
        else:
            pre = raw_ref[c] + selftab_ref[c]
            pre = pre / (s_ref[0] + 1.0)
        hc = jnp.maximum(pre + b_ref[pl.ds(c * 16, 16)], 0.0)
        acc = acc + jnp.dot(hc, w_ref[pl.ds(c * 16, 16), :],
                            preferred_element_type=F32, precision=jax.lax.Precision.DEFAULT)
    for c in range(4):
        xwtab_ref[c] = acc[:, c * 16:(c + 1) * 16]
    als = jnp.dot(acc, asrc_ref[...], preferred_element_type=F32, precision=jax.lax.Precision.DEFAULT)
    ald = jnp.dot(acc, adst_ref[...], preferred_element_type=F32, precision=jax.lax.Precision.DEFAULT)
    s = als + ald
    cs = jnp.maximum(s, 0.0) + 0.2 * jnp.minimum(s, 0.0)
    als_ref[...] = als
    ald_ref[...] = ald
    cs_ref[...] = cs


def _make_gatprep(scale_kind):
    sspec = (pl.BlockSpec((BLK, 1), lambda i: (i, 0)) if scale_kind == "dinv"
             else pl.BlockSpec((2, BLK, 1), lambda i: (0, i, 0)))
    return pl.pallas_call(
        functools.partial(_gatprep_body, scale_kind),
        grid=(NB,),
        in_specs=[
            pl.BlockSpec((4, BLK, 16), lambda i: (0, i, 0)),
            pl.BlockSpec((4, BLK, 16), lambda i: (0, i, 0)),
            sspec,
            pl.BlockSpec((64,), lambda i: (0,)),
            pl.BlockSpec((64, 64), lambda i: (0, 0)),
            pl.BlockSpec((64, 1), lambda i: (0, 0)),
            pl.BlockSpec((64, 1), lambda i: (0, 0)),
        ],
        out_specs=[
            pl.BlockSpec((4, BLK, 16), lambda i: (0, i, 0)),
            pl.BlockSpec((BLK, 1), lambda i: (i, 0)),
            pl.BlockSpec((BLK, 1), lambda i: (i, 0)),
            pl.BlockSpec((BLK, 1), lambda i: (i, 0)),
        ],
        out_shape=[
            jax.ShapeDtypeStruct((4, NPAD, 16), F32),
            jax.ShapeDtypeStruct((NPAD, 1), F32),
            jax.ShapeDtypeStruct((NPAD, 1), F32),
            jax.ShapeDtypeStruct((NPAD, 1), F32),
        ],
    )


_k_gatprep_dinv = _make_gatprep("dinv")
_k_gatprep_denom = _make_gatprep("denom")


def _final_body(raw_ref, selftab_ref, dp_ref, b_ref, batch_ref, wf1_ref,
                bf1_ref, wf2_ref, bf2_ref, out_ref, sums_ref, cnt_ref):
    i = pl.program_id(0)

    @pl.when(i == 0)
    def _():
        sums_ref[...] = jnp.zeros((G, 64), F32)
        cnt_ref[...] = jnp.zeros((G, 1), F32)

    denom = dp_ref[0] + 1.0
    hs = []
    for c in range(4):
        pre = (raw_ref[c] + selftab_ref[c]) / denom
        hs.append(jnp.maximum(pre + b_ref[pl.ds(c * 16, 16)], 0.0))
    h = jnp.concatenate(hs, axis=1)
    gids = lax.broadcasted_iota(I32, (1, G), 1)
    oh = (batch_ref[...] == gids).astype(F32)
    sums_ref[...] += lax.dot_general(oh, h, (((0,), (0,)), ((), ())),
                                     preferred_element_type=F32, precision=jax.lax.Precision.DEFAULT)
    cnt_ref[...] += lax.dot_general(oh, jnp.ones((BLK, 1), F32),
                                    (((0,), (0,)), ((), ())),
                                    preferred_element_type=F32, precision=jax.lax.Precision.DEFAULT)

    @pl.when(i == NB - 1)
    def _():
        pooled = sums_ref[...] / jnp.maximum(cnt_ref[...], 1.0)
        hf = jnp.maximum(
            jnp.dot(pooled, wf1_ref[...], preferred_element_type=F32, precision=jax.lax.Precision.DEFAULT)
            + bf1_ref[...], 0.0)
        out_ref[...] = (jnp.dot(hf, wf2_ref[...], preferred_element_type=F32, precision=jax.lax.Precision.DEFAULT)
                        + bf2_ref[...])


_k_final = pl.pallas_call(
    _final_body,
    grid=(NB,),
    in_specs=[
        pl.BlockSpec((4, BLK, 16), lambda i: (0, i, 0)),
        pl.BlockSpec((4, BLK, 16), lambda i: (0, i, 0)),
        pl.BlockSpec((2, BLK, 1), lambda i: (0, i, 0)),
        pl.BlockSpec((64,), lambda i: (0,)),
        pl.BlockSpec((BLK, 1), lambda i: (i, 0)),
        pl.BlockSpec((64, 32), lambda i: (0, 0)),
        pl.BlockSpec((32,), lambda i: (0,)),
        pl.BlockSpec((32, 1), lambda i: (0, 0)),
        pl.BlockSpec((1,), lambda i: (0,)),
    ],
    out_specs=pl.BlockSpec((G, 1), lambda i: (0, 0)),
    out_shape=jax.ShapeDtypeStruct((G, 1), F32),
    scratch_shapes=[
        pltpu.VMEM((G, 64), F32),
        pltpu.VMEM((G, 1), F32),
    ],
)


def kernel(x, edge_index, edge_attr, batch, W1, b1, W2, b2, Wg1, asrc1,
           adst1, bg1, Wg2, asrc2, adst2, bg2, Wf1, bf1, Wf2, bf2):
    del edge_attr
    npad_e = EPAD - edge_index.shape[1]
    padi = jnp.full((npad_e,), N, I32)
    src1 = jnp.concatenate([edge_index[0], padi])
    dst1 = jnp.concatenate([edge_index[1], padi])
    x_pad = jnp.zeros((NPAD, 32), F32).at[:N].set(x)
    batch_pad = jnp.full((NPAD, 1), -1, I32).at[:N, 0].set(batch)

    degp = _k_deg(dst1)
    ytab1, dinv = _k_prep1(x_pad, degp.reshape(2, NPAD, 1), W1)
    norm = _k_norm(dinv.reshape(NPAD), src1, dst1)
    raw1 = _k_rows_coef(ytab1.reshape(4 * NPAD, 16), src1, dst1, norm)
    ytab2 = _k_gcn1(raw1.reshape(4, NPAD, 16), ytab1, dinv, b1, W2)
    raw2 = _k_rows_coef(ytab2.reshape(4 * NPAD, 16), src1, dst1, norm)
    xwtab3, als3, ald3, cs3 = _k_gatprep_dinv(
        raw2.reshape(4, NPAD, 16), ytab2, dinv, b2, Wg1, asrc1.reshape(64, 1),
        adst1.reshape(64, 1))
    s3 = _k_score(als3.reshape(NPAD), ald3.reshape(NPAD),
                  cs3.reshape(NPAD), src1, dst1)
    ex3 = _k_exp(s3.reshape(_EXR, 128)).reshape(EPAD)
    raw3, dp3 = _k_rows_ex(xwtab3.reshape(4 * NPAD, 16), src1, dst1, ex3)
    xwtab4, als4, ald4, cs4 = _k_gatprep_denom(
        raw3.reshape(4, NPAD, 16), xwtab3, dp3.reshape(2, NPAD, 1), bg1, Wg2,
        asrc2.reshape(64, 1), adst2.reshape(64, 1))
    s4 = _k_score(als4.reshape(NPAD), ald4.reshape(NPAD),
                  cs4.reshape(NPAD), src1, dst1)
    ex4 = _k_exp(s4.reshape(_EXR, 128)).reshape(EPAD)
    raw4, dp4 = _k_rows_ex(xwtab4.reshape(4 * NPAD, 16), src1, dst1, ex4)
    out = _k_final(raw4.reshape(4, NPAD, 16), xwtab4, dp4.reshape(2, NPAD, 1),
                   bg2, batch_pad, Wf1, bf1, Wf2, bf2)
    return out.reshape(G)
